# baseline (speedup 1.0000x reference)
import functools

import jax
import jax.numpy as jnp
from jax import lax
from jax.experimental import pallas as pl
from jax.experimental.pallas import tpu as pltpu
from jax.experimental.pallas import tpu_sc as plsc

_N_PAD = 10240
_BLK_N = 256
_K = 8
_NW = 32
_CHUNK_NODES = 16


def _prep_body(cb_ref, w_ref, cbh_ref, csq_ref):
    cb = cb_ref[...]
    cbh_ref[...] = lax.dot_general(cb, w_ref[...], (((1,), (1,)), ((), ())),
                                   preferred_element_type=jnp.float32)
    sq = cb * cb
    ones = jnp.ones((1, cb.shape[1]), jnp.float32)
    csq_ref[...] = 0.5 * lax.dot_general(ones, sq, (((1,), (1,)), ((), ())),
                                         preferred_element_type=jnp.float32)


def _precompute(codebook, w_head):
    num_codes, d_in = codebook.shape
    d_out = w_head.shape[0]
    blk = 512
    grid = num_codes // blk
    return pl.pallas_call(
        _prep_body,
        grid=(grid,),
        in_specs=[
            pl.BlockSpec((blk, d_in), lambda i: (i, 0)),
            pl.BlockSpec((d_out, d_in), lambda i: (0, 0)),
        ],
        out_specs=[
            pl.BlockSpec((blk, d_out), lambda i: (i, 0)),
            pl.BlockSpec((1, blk), lambda i: (0, i)),
        ],
        out_shape=[
            jax.ShapeDtypeStruct((num_codes, d_out), jnp.float32),
            jax.ShapeDtypeStruct((1, num_codes), jnp.float32),
        ],
    )(codebook, w_head)


def _topk_body(x_ref, proj_ref, cb_ref, csq_ref, idx_ref, s_scr, *, k, grid):
    sx = s_scr[...]
    n, m = sx.shape
    iota = lax.broadcasted_iota(jnp.int32, (n, m), 1)
    big = jnp.int32(2**30)
    for j in range(k):
        mx = jnp.max(sx, axis=1, keepdims=True)
        ge = sx >= mx
        idx = jnp.min(jnp.where(ge, iota, big), axis=1, keepdims=True)
        idx_ref[:, j] = idx[:, 0]
        sx = jnp.where(ge, -jnp.inf, sx)

    h = jnp.dot(x_ref[...], proj_ref[...], preferred_element_type=jnp.float32)
    s = lax.dot_general(h, cb_ref[...], (((1,), (1,)), ((), ())),
                        preferred_element_type=jnp.float32)
    s_scr[...] = s - csq_ref[...]


def _topk_indices(x, proj, codebook, csq, k):
    n, d_in = x.shape
    num_codes = codebook.shape[0]
    grid = (n + _BLK_N - 1) // _BLK_N
    return pl.pallas_call(
        functools.partial(_topk_body, k=k, grid=grid),
        grid=(grid + 1,),
        in_specs=[
            pl.BlockSpec((_BLK_N, d_in), lambda i: (jnp.minimum(i, grid - 1), 0)),
            pl.BlockSpec((d_in, d_in), lambda i: (0, 0)),
            pl.BlockSpec((num_codes, d_in), lambda i: (0, 0)),
            pl.BlockSpec((1, num_codes), lambda i: (0, 0)),
        ],
        out_specs=pl.BlockSpec((_BLK_N, k), lambda i: (jnp.maximum(i, 1) - 1, 0)),
        out_shape=jax.ShapeDtypeStruct((n, k), jnp.int32),
        scratch_shapes=[pltpu.VMEM((_BLK_N, num_codes), jnp.float32)],
    )(x, proj, codebook, csq)


def _sc_gather_body(idx_hbm, cbh_hbm, bias_hbm, out_hbm,
                    idx_v, rows_v, acc_v, b_v, sem, *, npw, k):
    wid = lax.axis_index("s") * 2 + lax.axis_index("c")
    pltpu.sync_copy(bias_hbm, b_v)
    node_base = wid * npw
    n_chunks = npw // _CHUNK_NODES

    def chunk_body(ci, carry):
        nb = node_base + ci * _CHUNK_NODES
        pltpu.sync_copy(idx_hbm.at[pl.ds(nb * k, _CHUNK_NODES * k)], idx_v)
        pltpu.async_copy(cbh_hbm.at[idx_v], rows_v, sem).wait()

        def node_body(ni, c2):
            for c in range(8):
                sl = pl.ds(c * 16, 16)
                a = rows_v[ni * k, sl]
                for j in range(1, k):
                    a = a + rows_v[ni * k + j, sl]
                acc_v[ni, sl] = a * (1.0 / k) + b_v[sl]
            return c2

        lax.fori_loop(0, _CHUNK_NODES, node_body, 0)
        pltpu.sync_copy(acc_v, out_hbm.at[pl.ds(nb, _CHUNK_NODES)])
        return carry

    lax.fori_loop(0, n_chunks, chunk_body, 0)


def _sc_gather_mean(idx_flat, cb_head, b_head):
    d_out = cb_head.shape[1]
    n_half = idx_flat.shape[0] // _K
    npw = n_half // _NW
    mesh = plsc.VectorSubcoreMesh(core_axis_name="c", subcore_axis_name="s")
    kern = pl.kernel(
        functools.partial(_sc_gather_body, npw=npw, k=_K),
        out_type=jax.ShapeDtypeStruct((n_half, d_out), jnp.float32),
        mesh=mesh,
        scratch_types=[
            pltpu.VMEM((_CHUNK_NODES * _K,), jnp.int32),
            pltpu.VMEM((_CHUNK_NODES * _K, d_out), jnp.float32),
            pltpu.VMEM((_CHUNK_NODES, d_out), jnp.float32),
            pltpu.VMEM((d_out,), jnp.float32),
            pltpu.SemaphoreType.DMA,
        ],
    )
    return kern(idx_flat, cb_head, b_head)


def kernel(x, linear_proj, codebook, W_head, b_head):
    n = x.shape[0]
    cb_head, csq = _precompute(codebook, W_head)
    idx = _topk_indices(x, linear_proj, codebook, csq, _K)
    idx_pad = jnp.pad(idx, ((0, _N_PAD - n), (0, 0)))
    logits_pad = _sc_gather_mean(idx_pad.reshape(-1), cb_head, b_head)
    return logits_pad[:n]


# speedup vs baseline: 1.0680x; 1.0680x over previous
import functools

import jax
import jax.numpy as jnp
from jax import lax
from jax.experimental import pallas as pl
from jax.experimental.pallas import tpu as pltpu
from jax.experimental.pallas import tpu_sc as plsc

_N_PAD = 10240
_BLK_N = 256
_K = 8
_NW = 32
_CHUNK_NODES = 16


def _prep_body(cb_ref, w_ref, cbh_ref, csq_ref):
    cb = cb_ref[...]
    cbh_ref[...] = lax.dot_general(cb, w_ref[...], (((1,), (1,)), ((), ())),
                                   preferred_element_type=jnp.float32)
    sq = cb * cb
    ones = jnp.ones((1, cb.shape[1]), jnp.float32)
    csq_ref[...] = 0.5 * lax.dot_general(ones, sq, (((1,), (1,)), ((), ())),
                                         preferred_element_type=jnp.float32)


def _precompute(codebook, w_head):
    num_codes, d_in = codebook.shape
    d_out = w_head.shape[0]
    blk = 512
    grid = num_codes // blk
    return pl.pallas_call(
        _prep_body,
        grid=(grid,),
        in_specs=[
            pl.BlockSpec((blk, d_in), lambda i: (i, 0)),
            pl.BlockSpec((d_out, d_in), lambda i: (0, 0)),
        ],
        out_specs=[
            pl.BlockSpec((blk, d_out), lambda i: (i, 0)),
            pl.BlockSpec((1, blk), lambda i: (0, i)),
        ],
        out_shape=[
            jax.ShapeDtypeStruct((num_codes, d_out), jnp.float32),
            jax.ShapeDtypeStruct((1, num_codes), jnp.float32),
        ],
    )(codebook, w_head)


def _topk_body(x_ref, proj_ref, cb_ref, csq_ref, idx_ref, s_scr, *, k, grid):
    sx = s_scr[...]
    n, m = sx.shape
    iota = lax.broadcasted_iota(jnp.int32, (n, m), 1)
    big = jnp.int32(2**30)
    for j in range(k):
        mx = jnp.max(sx, axis=1, keepdims=True)
        ge = sx >= mx
        idx = jnp.min(jnp.where(ge, iota, big), axis=1, keepdims=True)
        idx_ref[:, j] = idx[:, 0]
        sx = jnp.where(ge, -jnp.inf, sx)

    h = jnp.dot(x_ref[...], proj_ref[...], preferred_element_type=jnp.float32)
    s = lax.dot_general(h, cb_ref[...], (((1,), (1,)), ((), ())),
                        preferred_element_type=jnp.float32)
    s_scr[...] = s - csq_ref[...]


def _topk_indices(x, proj, codebook, csq, k):
    n, d_in = x.shape
    num_codes = codebook.shape[0]
    grid = n // _BLK_N
    return pl.pallas_call(
        functools.partial(_topk_body, k=k, grid=grid),
        grid=(grid + 1,),
        in_specs=[
            pl.BlockSpec((_BLK_N, d_in), lambda i: (jnp.minimum(i, grid - 1), 0)),
            pl.BlockSpec((d_in, d_in), lambda i: (0, 0)),
            pl.BlockSpec((num_codes, d_in), lambda i: (0, 0)),
            pl.BlockSpec((1, num_codes), lambda i: (0, 0)),
        ],
        out_specs=pl.BlockSpec((_BLK_N, k), lambda i: (jnp.maximum(i, 1) - 1, 0)),
        out_shape=jax.ShapeDtypeStruct((n, k), jnp.int32),
        scratch_shapes=[pltpu.VMEM((_BLK_N, num_codes), jnp.float32)],
    )(x, proj, codebook, csq)


def _sc_gather_body(idx_hbm, cbh_hbm, bias_hbm, out_hbm,
                    idx_v, rows_v, acc_v, b_v, sem, *, npw, k):
    wid = lax.axis_index("s") * 2 + lax.axis_index("c")
    pltpu.sync_copy(bias_hbm, b_v)
    node_base = wid * npw
    n_chunks = npw // _CHUNK_NODES

    def chunk_body(ci, carry):
        nb = node_base + ci * _CHUNK_NODES
        pltpu.sync_copy(idx_hbm.at[pl.ds(nb * k, _CHUNK_NODES * k)], idx_v)
        pltpu.async_copy(cbh_hbm.at[idx_v], rows_v, sem).wait()

        def node_body(ni, c2):
            for c in range(8):
                sl = pl.ds(c * 16, 16)
                a = rows_v[ni * k, sl]
                for j in range(1, k):
                    a = a + rows_v[ni * k + j, sl]
                acc_v[ni, sl] = a * (1.0 / k) + b_v[sl]
            return c2

        lax.fori_loop(0, _CHUNK_NODES, node_body, 0)
        pltpu.sync_copy(acc_v, out_hbm.at[pl.ds(nb, _CHUNK_NODES)])
        return carry

    lax.fori_loop(0, n_chunks, chunk_body, 0)


def _sc_gather_mean(idx_flat, cb_head, b_head):
    d_out = cb_head.shape[1]
    n_half = idx_flat.shape[0] // _K
    npw = n_half // _NW
    mesh = plsc.VectorSubcoreMesh(core_axis_name="c", subcore_axis_name="s")
    kern = pl.kernel(
        functools.partial(_sc_gather_body, npw=npw, k=_K),
        out_type=jax.ShapeDtypeStruct((n_half, d_out), jnp.float32),
        mesh=mesh,
        scratch_types=[
            pltpu.VMEM((_CHUNK_NODES * _K,), jnp.int32),
            pltpu.VMEM((_CHUNK_NODES * _K, d_out), jnp.float32),
            pltpu.VMEM((_CHUNK_NODES, d_out), jnp.float32),
            pltpu.VMEM((d_out,), jnp.float32),
            pltpu.SemaphoreType.DMA,
        ],
    )
    return kern(idx_flat, cb_head, b_head)


def kernel(x, linear_proj, codebook, W_head, b_head):
    n = x.shape[0]
    cb_head, csq = _precompute(codebook, W_head)
    x_pad = jnp.pad(x, ((0, _N_PAD - n), (0, 0)))
    idx = _topk_indices(x_pad, linear_proj, codebook, csq, _K)
    logits_pad = _sc_gather_mean(idx.reshape(-1), cb_head, b_head)
    return logits_pad[:n]


# speedup vs baseline: 1.0849x; 1.0159x over previous
import functools

import jax
import jax.numpy as jnp
from jax import lax
from jax.experimental import pallas as pl
from jax.experimental.pallas import tpu as pltpu
from jax.experimental.pallas import tpu_sc as plsc

_N_PAD = 10240
_BLK_N = 256
_K = 8
_NW = 32
_CHUNK_NODES = 16


def _prep_body(cb_ref, w_ref, cbh_ref, csq_ref):
    cb = cb_ref[...]
    cbh_ref[...] = lax.dot_general(cb, w_ref[...], (((1,), (1,)), ((), ())),
                                   preferred_element_type=jnp.float32)
    sq = cb * cb
    ones = jnp.ones((1, cb.shape[1]), jnp.float32)
    csq_ref[...] = 0.5 * lax.dot_general(ones, sq, (((1,), (1,)), ((), ())),
                                         preferred_element_type=jnp.float32)


def _precompute(codebook, w_head):
    num_codes, d_in = codebook.shape
    d_out = w_head.shape[0]
    blk = 512
    grid = num_codes // blk
    return pl.pallas_call(
        _prep_body,
        grid=(grid,),
        in_specs=[
            pl.BlockSpec((blk, d_in), lambda i: (i, 0)),
            pl.BlockSpec((d_out, d_in), lambda i: (0, 0)),
        ],
        out_specs=[
            pl.BlockSpec((blk, d_out), lambda i: (i, 0)),
            pl.BlockSpec((1, blk), lambda i: (0, i)),
        ],
        out_shape=[
            jax.ShapeDtypeStruct((num_codes, d_out), jnp.float32),
            jax.ShapeDtypeStruct((1, num_codes), jnp.float32),
        ],
    )(codebook, w_head)


def _topk_body(x_ref, proj_ref, cb_ref, csq_ref, idx_ref, s_scr, *, k, grid):
    sx = s_scr[...]
    n, m = sx.shape
    iota = lax.broadcasted_iota(jnp.int32, (n, m), 1)
    big = jnp.int32(2**30)
    for j in range(k):
        mx = jnp.max(sx, axis=1, keepdims=True)
        ge = sx >= mx
        idx = jnp.min(jnp.where(ge, iota, big), axis=1, keepdims=True)
        idx_ref[:, j] = idx[:, 0]
        sx = jnp.where(ge, -jnp.inf, sx)

    h = jnp.dot(x_ref[...], proj_ref[...], preferred_element_type=jnp.float32)
    s = lax.dot_general(h, cb_ref[...], (((1,), (1,)), ((), ())),
                        preferred_element_type=jnp.float32)
    s_scr[...] = s - csq_ref[...]


def _topk_indices(x, proj, codebook, csq, k):
    n, d_in = x.shape
    num_codes = codebook.shape[0]
    grid = n // _BLK_N
    return pl.pallas_call(
        functools.partial(_topk_body, k=k, grid=grid),
        grid=(grid + 1,),
        in_specs=[
            pl.BlockSpec((_BLK_N, d_in), lambda i: (jnp.minimum(i, grid - 1), 0)),
            pl.BlockSpec((d_in, d_in), lambda i: (0, 0)),
            pl.BlockSpec((num_codes, d_in), lambda i: (0, 0)),
            pl.BlockSpec((1, num_codes), lambda i: (0, 0)),
        ],
        out_specs=pl.BlockSpec((_BLK_N, k), lambda i: (jnp.maximum(i, 1) - 1, 0)),
        out_shape=jax.ShapeDtypeStruct((n, k), jnp.int32),
        scratch_shapes=[pltpu.VMEM((_BLK_N, num_codes), jnp.float32)],
    )(x, proj, codebook, csq)


def _sc_gather_body(idx_hbm, cbh_hbm, bias_hbm, out_hbm,
                    idx_a, idx_b, rows_a, rows_b, acc_v, b_v,
                    sem_a, sem_b, *, npw, k):
    wid = lax.axis_index("s") * 2 + lax.axis_index("c")
    pltpu.sync_copy(bias_hbm, b_v)
    node_base = wid * npw
    n_pairs = npw // (2 * _CHUNK_NODES)
    bufs = ((idx_a, rows_a, sem_a), (idx_b, rows_b, sem_b))

    def accumulate(rows_v, nb):
        def node_body(ni, c2):
            for c in range(8):
                sl = pl.ds(c * 16, 16)
                a = rows_v[ni * k, sl]
                for j in range(1, k):
                    a = a + rows_v[ni * k + j, sl]
                acc_v[ni, sl] = a * (1.0 / k) + b_v[sl]
            return c2

        lax.fori_loop(0, _CHUNK_NODES, node_body, 0)
        pltpu.sync_copy(acc_v, out_hbm.at[pl.ds(nb, _CHUNK_NODES)])

    def pair_body(p, carry):
        copies = []
        for b, (idx_v, rows_v, sem) in enumerate(bufs):
            nb = node_base + (p * 2 + b) * _CHUNK_NODES
            pltpu.sync_copy(idx_hbm.at[pl.ds(nb * k, _CHUNK_NODES * k)], idx_v)
            copies.append(pltpu.async_copy(cbh_hbm.at[idx_v], rows_v, sem))
        for b, (idx_v, rows_v, sem) in enumerate(bufs):
            nb = node_base + (p * 2 + b) * _CHUNK_NODES
            copies[b].wait()
            accumulate(rows_v, nb)
        return carry

    lax.fori_loop(0, n_pairs, pair_body, 0)


def _sc_gather_mean(idx_flat, cb_head, b_head):
    d_out = cb_head.shape[1]
    n_half = idx_flat.shape[0] // _K
    npw = n_half // _NW
    mesh = plsc.VectorSubcoreMesh(core_axis_name="c", subcore_axis_name="s")
    kern = pl.kernel(
        functools.partial(_sc_gather_body, npw=npw, k=_K),
        out_type=jax.ShapeDtypeStruct((n_half, d_out), jnp.float32),
        mesh=mesh,
        scratch_types=[
            pltpu.VMEM((_CHUNK_NODES * _K,), jnp.int32),
            pltpu.VMEM((_CHUNK_NODES * _K,), jnp.int32),
            pltpu.VMEM((_CHUNK_NODES * _K, d_out), jnp.float32),
            pltpu.VMEM((_CHUNK_NODES * _K, d_out), jnp.float32),
            pltpu.VMEM((_CHUNK_NODES, d_out), jnp.float32),
            pltpu.VMEM((d_out,), jnp.float32),
            pltpu.SemaphoreType.DMA,
            pltpu.SemaphoreType.DMA,
        ],
    )
    return kern(idx_flat, cb_head, b_head)


def kernel(x, linear_proj, codebook, W_head, b_head):
    n = x.shape[0]
    cb_head, csq = _precompute(codebook, W_head)
    x_pad = jnp.pad(x, ((0, _N_PAD - n), (0, 0)))
    idx = _topk_indices(x_pad, linear_proj, codebook, csq, _K)
    logits_pad = _sc_gather_mean(idx.reshape(-1), cb_head, b_head)
    return logits_pad[:n]


# speedup vs baseline: 1.0985x; 1.0124x over previous
import functools

import jax
import jax.numpy as jnp
from jax import lax
from jax.experimental import pallas as pl
from jax.experimental.pallas import tpu as pltpu
from jax.experimental.pallas import tpu_sc as plsc

_N_PAD = 10240
_BLK_N = 512
_K = 8
_NW = 32
_CHUNK_NODES = 16


def _prep_body(cb_ref, w_ref, cbh_ref, csq_ref):
    cb = cb_ref[...]
    cbh_ref[...] = lax.dot_general(cb, w_ref[...], (((1,), (1,)), ((), ())),
                                   preferred_element_type=jnp.float32)
    sq = cb * cb
    ones = jnp.ones((1, cb.shape[1]), jnp.float32)
    csq_ref[...] = 0.5 * lax.dot_general(ones, sq, (((1,), (1,)), ((), ())),
                                         preferred_element_type=jnp.float32)


def _precompute(codebook, w_head):
    num_codes, d_in = codebook.shape
    d_out = w_head.shape[0]
    blk = 512
    grid = num_codes // blk
    return pl.pallas_call(
        _prep_body,
        grid=(grid,),
        in_specs=[
            pl.BlockSpec((blk, d_in), lambda i: (i, 0)),
            pl.BlockSpec((d_out, d_in), lambda i: (0, 0)),
        ],
        out_specs=[
            pl.BlockSpec((blk, d_out), lambda i: (i, 0)),
            pl.BlockSpec((1, blk), lambda i: (0, i)),
        ],
        out_shape=[
            jax.ShapeDtypeStruct((num_codes, d_out), jnp.float32),
            jax.ShapeDtypeStruct((1, num_codes), jnp.float32),
        ],
    )(codebook, w_head)


def _topk_body(x_ref, proj_ref, cb_ref, csq_ref, idx_ref, s_scr, *, k, grid):
    sx = s_scr[...]
    n, m = sx.shape
    iota = lax.broadcasted_iota(jnp.int32, (n, m), 1)
    big = jnp.int32(2**30)
    for j in range(k):
        mx = jnp.max(sx, axis=1, keepdims=True)
        ge = sx >= mx
        idx = jnp.min(jnp.where(ge, iota, big), axis=1, keepdims=True)
        idx_ref[:, j] = idx[:, 0]
        sx = jnp.where(ge, -jnp.inf, sx)

    h = jnp.dot(x_ref[...], proj_ref[...], preferred_element_type=jnp.float32)
    s = lax.dot_general(h, cb_ref[...], (((1,), (1,)), ((), ())),
                        preferred_element_type=jnp.float32)
    s_scr[...] = s - csq_ref[...]


def _topk_indices(x, proj, codebook, csq, k):
    n, d_in = x.shape
    num_codes = codebook.shape[0]
    grid = n // _BLK_N
    return pl.pallas_call(
        functools.partial(_topk_body, k=k, grid=grid),
        grid=(grid + 1,),
        in_specs=[
            pl.BlockSpec((_BLK_N, d_in), lambda i: (jnp.minimum(i, grid - 1), 0)),
            pl.BlockSpec((d_in, d_in), lambda i: (0, 0)),
            pl.BlockSpec((num_codes, d_in), lambda i: (0, 0)),
            pl.BlockSpec((1, num_codes), lambda i: (0, 0)),
        ],
        out_specs=pl.BlockSpec((_BLK_N, k), lambda i: (jnp.maximum(i, 1) - 1, 0)),
        out_shape=jax.ShapeDtypeStruct((n, k), jnp.int32),
        scratch_shapes=[pltpu.VMEM((_BLK_N, num_codes), jnp.float32)],
    )(x, proj, codebook, csq)


def _sc_gather_body(idx_hbm, cbh_hbm, bias_hbm, out_hbm,
                    idx_a, idx_b, rows_a, rows_b, acc_v, b_v,
                    sem_a, sem_b, *, npw, k):
    wid = lax.axis_index("s") * 2 + lax.axis_index("c")
    pltpu.sync_copy(bias_hbm, b_v)
    node_base = wid * npw
    n_pairs = npw // (2 * _CHUNK_NODES)
    bufs = ((idx_a, rows_a, sem_a), (idx_b, rows_b, sem_b))

    def accumulate(rows_v, nb):
        def node_body(ni, c2):
            for c in range(8):
                sl = pl.ds(c * 16, 16)
                a = rows_v[ni * k, sl]
                for j in range(1, k):
                    a = a + rows_v[ni * k + j, sl]
                acc_v[ni, sl] = a * (1.0 / k) + b_v[sl]
            return c2

        lax.fori_loop(0, _CHUNK_NODES, node_body, 0)
        pltpu.sync_copy(acc_v, out_hbm.at[pl.ds(nb, _CHUNK_NODES)])

    def pair_body(p, carry):
        copies = []
        for b, (idx_v, rows_v, sem) in enumerate(bufs):
            nb = node_base + (p * 2 + b) * _CHUNK_NODES
            pltpu.sync_copy(idx_hbm.at[pl.ds(nb * k, _CHUNK_NODES * k)], idx_v)
            copies.append(pltpu.async_copy(cbh_hbm.at[idx_v], rows_v, sem))
        for b, (idx_v, rows_v, sem) in enumerate(bufs):
            nb = node_base + (p * 2 + b) * _CHUNK_NODES
            copies[b].wait()
            accumulate(rows_v, nb)
        return carry

    lax.fori_loop(0, n_pairs, pair_body, 0)


def _sc_gather_mean(idx_flat, cb_head, b_head):
    d_out = cb_head.shape[1]
    n_half = idx_flat.shape[0] // _K
    npw = n_half // _NW
    mesh = plsc.VectorSubcoreMesh(core_axis_name="c", subcore_axis_name="s")
    kern = pl.kernel(
        functools.partial(_sc_gather_body, npw=npw, k=_K),
        out_type=jax.ShapeDtypeStruct((n_half, d_out), jnp.float32),
        mesh=mesh,
        scratch_types=[
            pltpu.VMEM((_CHUNK_NODES * _K,), jnp.int32),
            pltpu.VMEM((_CHUNK_NODES * _K,), jnp.int32),
            pltpu.VMEM((_CHUNK_NODES * _K, d_out), jnp.float32),
            pltpu.VMEM((_CHUNK_NODES * _K, d_out), jnp.float32),
            pltpu.VMEM((_CHUNK_NODES, d_out), jnp.float32),
            pltpu.VMEM((d_out,), jnp.float32),
            pltpu.SemaphoreType.DMA,
            pltpu.SemaphoreType.DMA,
        ],
    )
    return kern(idx_flat, cb_head, b_head)


def kernel(x, linear_proj, codebook, W_head, b_head):
    n = x.shape[0]
    cb_head, csq = _precompute(codebook, W_head)
    x_pad = jnp.pad(x, ((0, _N_PAD - n), (0, 0)))
    idx = _topk_indices(x_pad, linear_proj, codebook, csq, _K)
    logits_pad = _sc_gather_mean(idx.reshape(-1), cb_head, b_head)
    return logits_pad[:n]
